# baseline (speedup 1.0000x reference)
import jax
import jax.numpy as jnp
from jax import lax
from jax.experimental import pallas as pl
from jax.experimental.pallas import tpu as pltpu
from jax.experimental.pallas import tpu_sc as plsc

_NS = 16
_NC = 2
_CH = 128
_EPC = _CH
_L = 16


def _seg_sum_call(table, srcp, dstp, NP):
  W = table.shape[1]
  NCH = srcp.shape[2]
  rpt = NP // _NS
  mesh = plsc.VectorSubcoreMesh(core_axis_name="c", subcore_axis_name="s")

  def body(table_h, src_h, dst_h, acc_out, src_v, dst_v, rows_v, zrow_v,
           acc_s, sem):
    cid = lax.axis_index("c")
    sid = lax.axis_index("s")
    base = sid * rpt

    for i in range(16):
      for j in range(W // _L):
        zrow_v[i, pl.ds(j * _L, _L)] = jnp.zeros((_L,), jnp.float32)

    def zloop(r, _):
      pltpu.sync_copy(zrow_v, acc_s.at[pl.ds(base + r * 16, 16)])
      return 0
    lax.fori_loop(0, rpt // 16, zloop, 0)
    plsc.subcore_barrier()

    def chunk(c, _):
      pltpu.sync_copy(src_h.at[cid, sid, c], src_v)
      pltpu.sync_copy(dst_h.at[cid, sid, c], dst_v)
      pltpu.async_copy(table_h.at[src_v], rows_v, sem).wait()
      pltpu.sync_copy(rows_v, acc_s.at[dst_v], add=True)
      return 0
    lax.fori_loop(0, NCH, chunk, 0)
    plsc.subcore_barrier()

    pltpu.sync_copy(acc_s.at[pl.ds(base, rpt)],
                    acc_out.at[cid, pl.ds(base, rpt)])

  fn = pl.kernel(
      body,
      out_type=jax.ShapeDtypeStruct((2, NP, W), jnp.float32),
      mesh=mesh,
      scratch_types=(
          pltpu.VMEM(srcp.shape[3:], jnp.int32),
          pltpu.VMEM(srcp.shape[3:], jnp.int32),
          pltpu.VMEM((_EPC, W), jnp.float32),

          pltpu.VMEM((16, W), jnp.float32),
          pltpu.VMEM_SHARED((NP, W), jnp.float32),
          pltpu.SemaphoreType.DMA,
      ),
  )
  return fn(table, srcp, dstp)


def _count_call(sd, NP):
  NO = sd.shape[2]
  mesh = plsc.VectorSubcoreMesh(core_axis_name="c", subcore_axis_name="s")

  def body(sd_h, cnt_out, idx_v, cnt_v):
    cid = lax.axis_index("c")
    sid = lax.axis_index("s")

    def z1(r, _):
      cnt_v[pl.ds(r * _L, _L)] = jnp.zeros((_L,), jnp.float32)
      return 0
    lax.fori_loop(0, NP // _L, z1, 0)

    ones16 = jnp.ones((_L,), jnp.float32)

    def octet(ob, _):
      pltpu.sync_copy(sd_h.at[cid, sid, ob], idx_v)
      for j in range(8):
        for k in range(_CH // _L):
          idxv = idx_v[j, 1, pl.ds(k * _L, _L)]
          plsc.addupdate_scatter(cnt_v, [idxv], ones16)
      return 0
    lax.fori_loop(0, NO, octet, 0)

    pltpu.sync_copy(cnt_v, cnt_out.at[cid, sid])

  fn = pl.kernel(
      body,
      out_type=jax.ShapeDtypeStruct((2, _NS, NP), jnp.float32),
      mesh=mesh,
      compiler_params=pltpu.CompilerParams(needs_layout_passes=False),
      scratch_types=(
          pltpu.VMEM((8, 2, _CH), jnp.int32),
          pltpu.VMEM((NP,), jnp.float32),
      ),
  )
  return fn(sd)


def _tc_layer1(acc, cnt, xin, wlT, bl, wrT, R=1024):
  G, NP, D = acc.shape

  def body(acc_r, cnt_r, x_r, wl_r, bl_r, wr_r, out_r):
    c = jnp.maximum(jnp.sum(cnt_r[0], axis=0), 1.0)[:, None]
    h = (jnp.dot(acc_r[0] / c, wl_r[...], preferred_element_type=jnp.float32)
         + jnp.dot(x_r[0], wr_r[...], preferred_element_type=jnp.float32)
         + bl_r[...])
    out_r[0] = jnp.maximum(h, 0.0)

  return pl.pallas_call(
      body,
      grid=(G, NP // R),
      in_specs=[
          pl.BlockSpec((1, R, D), lambda g, i: (g, i, 0)),
          pl.BlockSpec((1, _NS, R), lambda g, i: (g, 0, i)),
          pl.BlockSpec((1, R, D), lambda g, i: (g, i, 0)),
          pl.BlockSpec((D, D), lambda g, i: (0, 0)),
          pl.BlockSpec((1, D), lambda g, i: (0, 0)),
          pl.BlockSpec((D, D), lambda g, i: (0, 0)),
      ],
      out_specs=pl.BlockSpec((1, R, D), lambda g, i: (g, i, 0)),
      out_shape=jax.ShapeDtypeStruct((G, NP, D), jnp.float32),
  )(acc, cnt, xin, wlT, bl, wrT)


def _tc_layer2(acc, cnt, xz1, wlT, bl, wrT, w8, b8, R=1024):
  G, NP, D = acc.shape

  def body(acc_r, cnt_r, x_r, wl_r, bl_r, wr_r, w8_r, b8_r, out_r):
    c = jnp.maximum(jnp.sum(cnt_r[0], axis=0), 1.0)[:, None]
    xz2 = (jnp.dot(acc_r[0] / c, wl_r[...], preferred_element_type=jnp.float32)
           + jnp.dot(x_r[0], wr_r[...], preferred_element_type=jnp.float32)
           + bl_r[...])
    hd = lax.dot_general(w8_r[...], xz2, (((1,), (1,)), ((), ())),
                         preferred_element_type=jnp.float32) + b8_r[...]
    out_r[0] = jnp.where(hd >= 0.0, hd, 0.01 * hd)

  return pl.pallas_call(
      body,
      grid=(G, NP // R),
      in_specs=[
          pl.BlockSpec((1, R, D), lambda g, i: (g, i, 0)),
          pl.BlockSpec((1, _NS, R), lambda g, i: (g, 0, i)),
          pl.BlockSpec((1, R, D), lambda g, i: (g, i, 0)),
          pl.BlockSpec((D, D), lambda g, i: (0, 0)),
          pl.BlockSpec((1, D), lambda g, i: (0, 0)),
          pl.BlockSpec((D, D), lambda g, i: (0, 0)),
          pl.BlockSpec((8, D), lambda g, i: (0, 0)),
          pl.BlockSpec((8, 1), lambda g, i: (0, 0)),
      ],
      out_specs=pl.BlockSpec((1, 8, R), lambda g, i: (g, 0, i)),
      out_shape=jax.ShapeDtypeStruct((G, 8, NP), jnp.float32),
  )(acc, cnt, xz1, wlT, bl, wrT, w8, b8)


def _gather_heads(hv, maskp, tp, cp):
  H4, NPad = hv.shape
  MP = maskp.shape[0]
  TP = tp.shape[0]
  mesh = plsc.VectorSubcoreMesh(core_axis_name="c", subcore_axis_name="s")

  def body(hv_h, mask_h, t_h, c_h, out_h, hv_v, mask_v, t_v, c_v, out_v):
    cid = lax.axis_index("c")
    sid = lax.axis_index("s")

    pltpu.sync_copy(hv_h, hv_v)
    pltpu.sync_copy(mask_h, mask_v)
    pltpu.sync_copy(t_h, t_v)
    pltpu.sync_copy(c_h, c_v)
    for i in range(TP // _L):
      sl = pl.ds(i * _L, _L)
      rt = plsc.load_gather(mask_v, [t_v[sl]])
      rc = plsc.load_gather(mask_v, [c_v[sl]])
      for k, r in ((0, rt), (1, rt), (2, rc), (3, rc)):
        row = jnp.full((_L,), k, jnp.int32)
        out_v[k, sl] = plsc.load_gather(hv_v, [row, r])

    @pl.when(jnp.logical_and(cid == 0, sid == 0))
    def _():
      pltpu.sync_copy(out_v, out_h)

  fn = pl.kernel(
      body,
      out_type=jax.ShapeDtypeStruct((4, TP), jnp.float32),
      mesh=mesh,
      compiler_params=pltpu.CompilerParams(needs_layout_passes=False),
      scratch_types=(
          pltpu.VMEM((H4, NPad), jnp.float32),
          pltpu.VMEM((MP,), jnp.int32),
          pltpu.VMEM((TP,), jnp.int32),
          pltpu.VMEM((TP,), jnp.int32),
          pltpu.VMEM((4, TP), jnp.float32),
      ),
  )
  return fn(hv, maskp, tp, cp)


def kernel(x, edge_index, fake_x, fake_edge_index, mask, treat_idx, control_idx,
           Wl1, bl1, Wr1, Wl2, bl2, Wr2, Wy1, by1, Wy0, by0, Wbal, bbal,
           Wprop, bprop):
  N, D = x.shape
  E = edge_index.shape[1]
  f32 = jnp.float32
  i32 = jnp.int32

  NP = -(-(N + 1) // 256) * 256
  OCT = 8 * _CH
  EPT = -(-E // _NS // OCT) * OCT
  EP = EPT * _NS
  NO = EPT // OCT
  NCH = EPT // _EPC

  xpad = jnp.zeros((2 * NP, D), f32).at[:N].set(x).at[NP:NP + N].set(fake_x)

  pad_src = jnp.zeros((EP - E,), i32)
  pad_dst = jnp.full((EP - E,), N, i32)
  src2 = jnp.stack([jnp.concatenate([edge_index[0], pad_src]),
                    jnp.concatenate([fake_edge_index[0] + NP, pad_src])])
  dst2 = jnp.stack([jnp.concatenate([edge_index[1], pad_dst]),
                    jnp.concatenate([fake_edge_index[1], pad_dst])])

  srcp = src2.reshape(2, _NS, NCH, _CH)
  dstp = dst2.reshape(2, _NS, NCH, _CH)
  sd = jnp.concatenate([src2.reshape(2, _NS, NO, 8, 1, _CH),
                        dst2.reshape(2, _NS, NO, 8, 1, _CH)],
                       axis=4)

  acc1 = _seg_sum_call(xpad, srcp, dstp, NP)
  cnt = _count_call(sd, NP)
  xz1 = _tc_layer1(acc1, cnt, xpad.reshape(2, NP, D),
                   Wl1.T, bl1.reshape(1, D), Wr1.T)
  acc2 = _seg_sum_call(xz1.reshape(2 * NP, D), srcp, dstp, NP)

  w8 = jnp.concatenate([Wy1, Wy0, Wprop, Wbal, jnp.zeros((4, D), f32)], 0)
  b8 = jnp.concatenate([by1, by0, bprop, bbal,
                        jnp.zeros((4,), f32)]).reshape(8, 1)
  heads = _tc_layer2(acc2, cnt, xz1, Wl2.T, bl2.reshape(1, D), Wr2.T, w8, b8)

  hv = jnp.stack([heads[0, 0], heads[1, 0], heads[0, 1], heads[1, 1]])

  T = treat_idx.shape[0]
  M = mask.shape[0]
  TP = -(-T // _CH) * _CH
  MP = -(-M // _CH) * _CH
  maskp = jnp.concatenate([mask, jnp.zeros((MP - M,), i32)])
  tpad = jnp.concatenate([treat_idx, jnp.zeros((TP - T,), i32)])
  cpad = jnp.concatenate([control_idx, jnp.zeros((TP - T,), i32)])
  g4 = _gather_heads(hv, maskp, tpad, cpad)

  return (g4[0, :T], g4[1, :T], g4[2, :T], g4[3, :T],
          heads[0, 2, :N], heads[1, 2, :N], heads[0, 3, :N])


# speedup vs baseline: 1.0058x; 1.0058x over previous
import jax
import jax.numpy as jnp
from jax import lax
from jax.experimental import pallas as pl
from jax.experimental.pallas import tpu as pltpu
from jax.experimental.pallas import tpu_sc as plsc

_NS = 16
_NC = 2
_CH = 128
_EPC = _CH
_L = 16


def _seg_sum_call(table, srcp, dstp, NP):
  W = table.shape[1]
  NCH = srcp.shape[2]
  rpt = NP // _NS
  mesh = plsc.VectorSubcoreMesh(core_axis_name="c", subcore_axis_name="s")

  def body(table_h, src_h, dst_h, acc_out, src_v, dst_v, rows_v, zrow_v,
           acc_s, sem):
    cid = lax.axis_index("c")
    sid = lax.axis_index("s")
    base = sid * rpt

    for i in range(16):
      for j in range(W // _L):
        zrow_v[i, pl.ds(j * _L, _L)] = jnp.zeros((_L,), jnp.float32)

    def zloop(r, _):
      pltpu.sync_copy(zrow_v, acc_s.at[pl.ds(base + r * 16, 16)])
      return 0
    lax.fori_loop(0, rpt // 16, zloop, 0)
    plsc.subcore_barrier()

    def chunk(c, _):
      pltpu.sync_copy(src_h.at[cid, sid, c], src_v)
      pltpu.sync_copy(dst_h.at[cid, sid, c], dst_v)
      pltpu.async_copy(table_h.at[src_v], rows_v, sem).wait()
      pltpu.sync_copy(rows_v, acc_s.at[dst_v], add=True)
      return 0
    lax.fori_loop(0, NCH, chunk, 0)
    plsc.subcore_barrier()

    pltpu.sync_copy(acc_s.at[pl.ds(base, rpt)],
                    acc_out.at[cid, pl.ds(base, rpt)])

  fn = pl.kernel(
      body,
      out_type=jax.ShapeDtypeStruct((2, NP, W), jnp.float32),
      mesh=mesh,
      scratch_types=(
          pltpu.VMEM(srcp.shape[3:], jnp.int32),
          pltpu.VMEM(srcp.shape[3:], jnp.int32),
          pltpu.VMEM((_EPC, W), jnp.float32),

          pltpu.VMEM((16, W), jnp.float32),
          pltpu.VMEM_SHARED((NP, W), jnp.float32),
          pltpu.SemaphoreType.DMA,
      ),
  )
  return fn(table, srcp, dstp)


def _count_call(sd, NP):
  NO = sd.shape[2]
  mesh = plsc.VectorSubcoreMesh(core_axis_name="c", subcore_axis_name="s")

  def body(sd_h, cnt_out, idx_v, cnt_v):
    cid = lax.axis_index("c")
    sid = lax.axis_index("s")

    def z1(r, _):
      cnt_v[pl.ds(r * _L, _L)] = jnp.zeros((_L,), jnp.float32)
      return 0
    lax.fori_loop(0, NP // _L, z1, 0)

    ones16 = jnp.ones((_L,), jnp.float32)

    def octet(ob, _):
      pltpu.sync_copy(sd_h.at[cid, sid, ob], idx_v)
      for j in range(8):
        for k in range(_CH // _L):
          idxv = idx_v[j, 1, pl.ds(k * _L, _L)]
          plsc.addupdate_scatter(cnt_v, [idxv], ones16)
      return 0
    lax.fori_loop(0, NO, octet, 0)

    pltpu.sync_copy(cnt_v, cnt_out.at[cid, sid])

  fn = pl.kernel(
      body,
      out_type=jax.ShapeDtypeStruct((2, _NS, NP), jnp.float32),
      mesh=mesh,
      compiler_params=pltpu.CompilerParams(needs_layout_passes=False),
      scratch_types=(
          pltpu.VMEM((8, 2, _CH), jnp.int32),
          pltpu.VMEM((NP,), jnp.float32),
      ),
  )
  return fn(sd)


def _tc_layer1(acc, cnt, xin, wlT, bl, wrT, R=1024):
  G, NP, D = acc.shape

  def body(acc_r, cnt_r, x_r, wl_r, bl_r, wr_r, out_r):
    c = jnp.maximum(jnp.sum(cnt_r[0], axis=0), 1.0)[:, None]
    h = (jnp.dot(acc_r[0] / c, wl_r[...], preferred_element_type=jnp.float32)
         + jnp.dot(x_r[0], wr_r[...], preferred_element_type=jnp.float32)
         + bl_r[...])
    out_r[0] = jnp.maximum(h, 0.0)

  return pl.pallas_call(
      body,
      grid=(G, NP // R),
      in_specs=[
          pl.BlockSpec((1, R, D), lambda g, i: (g, i, 0)),
          pl.BlockSpec((1, _NS, R), lambda g, i: (g, 0, i)),
          pl.BlockSpec((1, R, D), lambda g, i: (g, i, 0)),
          pl.BlockSpec((D, D), lambda g, i: (0, 0)),
          pl.BlockSpec((1, D), lambda g, i: (0, 0)),
          pl.BlockSpec((D, D), lambda g, i: (0, 0)),
      ],
      out_specs=pl.BlockSpec((1, R, D), lambda g, i: (g, i, 0)),
      out_shape=jax.ShapeDtypeStruct((G, NP, D), jnp.float32),
  )(acc, cnt, xin, wlT, bl, wrT)


def _tc_layer2(acc, cnt, xz1, wlT, bl, wrT, w8, b8, R=1024):
  G, NP, D = acc.shape

  def body(acc_r, cnt_r, x_r, wl_r, bl_r, wr_r, w8_r, b8_r, out_r):
    c = jnp.maximum(jnp.sum(cnt_r[0], axis=0), 1.0)[:, None]
    xz2 = (jnp.dot(acc_r[0] / c, wl_r[...], preferred_element_type=jnp.float32)
           + jnp.dot(x_r[0], wr_r[...], preferred_element_type=jnp.float32)
           + bl_r[...])
    hd = lax.dot_general(w8_r[...], xz2, (((1,), (1,)), ((), ())),
                         preferred_element_type=jnp.float32) + b8_r[...]
    out_r[0] = jnp.where(hd >= 0.0, hd, 0.01 * hd)

  return pl.pallas_call(
      body,
      grid=(G, NP // R),
      in_specs=[
          pl.BlockSpec((1, R, D), lambda g, i: (g, i, 0)),
          pl.BlockSpec((1, _NS, R), lambda g, i: (g, 0, i)),
          pl.BlockSpec((1, R, D), lambda g, i: (g, i, 0)),
          pl.BlockSpec((D, D), lambda g, i: (0, 0)),
          pl.BlockSpec((1, D), lambda g, i: (0, 0)),
          pl.BlockSpec((D, D), lambda g, i: (0, 0)),
          pl.BlockSpec((8, D), lambda g, i: (0, 0)),
          pl.BlockSpec((8, 1), lambda g, i: (0, 0)),
      ],
      out_specs=pl.BlockSpec((1, 8, R), lambda g, i: (g, 0, i)),
      out_shape=jax.ShapeDtypeStruct((G, 8, NP), jnp.float32),
  )(acc, cnt, xz1, wlT, bl, wrT, w8, b8)


def _gather_heads(hv, maskp, tp, cp):
  H4, NPad = hv.shape
  MP = maskp.shape[0]
  TP = tp.shape[0]
  mesh = plsc.VectorSubcoreMesh(core_axis_name="c", subcore_axis_name="s")

  def body(hv_h, mask_h, t_h, c_h, out_h, hv_v, mask_v, t_v, c_v, out_v):
    cid = lax.axis_index("c")
    sid = lax.axis_index("s")

    pltpu.sync_copy(hv_h, hv_v)
    pltpu.sync_copy(mask_h, mask_v)
    pltpu.sync_copy(t_h, t_v)
    pltpu.sync_copy(c_h, c_v)
    for i in range(TP // _L):
      sl = pl.ds(i * _L, _L)
      rt = plsc.load_gather(mask_v, [t_v[sl]])
      rc = plsc.load_gather(mask_v, [c_v[sl]])
      for k, r in ((0, rt), (1, rt), (2, rc), (3, rc)):
        row = jnp.full((_L,), k, jnp.int32)
        out_v[k, sl] = plsc.load_gather(hv_v, [row, r])

    @pl.when(jnp.logical_and(cid == 0, sid == 0))
    def _():
      pltpu.sync_copy(out_v, out_h)

  fn = pl.kernel(
      body,
      out_type=jax.ShapeDtypeStruct((4, TP), jnp.float32),
      mesh=mesh,
      compiler_params=pltpu.CompilerParams(needs_layout_passes=False),
      scratch_types=(
          pltpu.VMEM((H4, NPad), jnp.float32),
          pltpu.VMEM((MP,), jnp.int32),
          pltpu.VMEM((TP,), jnp.int32),
          pltpu.VMEM((TP,), jnp.int32),
          pltpu.VMEM((4, TP), jnp.float32),
      ),
  )
  return fn(hv, maskp, tp, cp)


def kernel(x, edge_index, fake_x, fake_edge_index, mask, treat_idx, control_idx,
           Wl1, bl1, Wr1, Wl2, bl2, Wr2, Wy1, by1, Wy0, by0, Wbal, bbal,
           Wprop, bprop):
  N, D = x.shape
  E = edge_index.shape[1]
  f32 = jnp.float32
  i32 = jnp.int32

  NP = -(-(N + 1) // 256) * 256
  OCT = 8 * _CH
  EPT = -(-E // _NS // OCT) * OCT
  EP = EPT * _NS
  NO = EPT // OCT
  NCH = EPT // _EPC

  xpad = jnp.zeros((2 * NP, D), f32).at[:N].set(x).at[NP:NP + N].set(fake_x)

  pad_src = jnp.zeros((EP - E,), i32)
  pad_dst = N + (jnp.arange(EP - E, dtype=i32) % (NP - N))
  src2 = jnp.stack([jnp.concatenate([edge_index[0], pad_src]),
                    jnp.concatenate([fake_edge_index[0] + NP, pad_src])])
  dst2 = jnp.stack([jnp.concatenate([edge_index[1], pad_dst]),
                    jnp.concatenate([fake_edge_index[1], pad_dst])])

  srcp = src2.reshape(2, _NS, NCH, _CH)
  dstp = dst2.reshape(2, _NS, NCH, _CH)
  sd = jnp.concatenate([src2.reshape(2, _NS, NO, 8, 1, _CH),
                        dst2.reshape(2, _NS, NO, 8, 1, _CH)],
                       axis=4)

  acc1 = _seg_sum_call(xpad, srcp, dstp, NP)
  cnt = _count_call(sd, NP)
  xz1 = _tc_layer1(acc1, cnt, xpad.reshape(2, NP, D),
                   Wl1.T, bl1.reshape(1, D), Wr1.T)
  acc2 = _seg_sum_call(xz1.reshape(2 * NP, D), srcp, dstp, NP)

  w8 = jnp.concatenate([Wy1, Wy0, Wprop, Wbal, jnp.zeros((4, D), f32)], 0)
  b8 = jnp.concatenate([by1, by0, bprop, bbal,
                        jnp.zeros((4,), f32)]).reshape(8, 1)
  heads = _tc_layer2(acc2, cnt, xz1, Wl2.T, bl2.reshape(1, D), Wr2.T, w8, b8)

  hv = jnp.stack([heads[0, 0], heads[1, 0], heads[0, 1], heads[1, 1]])

  T = treat_idx.shape[0]
  M = mask.shape[0]
  TP = -(-T // _CH) * _CH
  MP = -(-M // _CH) * _CH
  maskp = jnp.concatenate([mask, jnp.zeros((MP - M,), i32)])
  tpad = jnp.concatenate([treat_idx, jnp.zeros((TP - T,), i32)])
  cpad = jnp.concatenate([control_idx, jnp.zeros((TP - T,), i32)])
  g4 = _gather_heads(hv, maskp, tpad, cpad)

  return (g4[0, :T], g4[1, :T], g4[2, :T], g4[3, :T],
          heads[0, 2, :N], heads[1, 2, :N], heads[0, 3, :N])


# speedup vs baseline: 1.5419x; 1.5330x over previous
import jax
import jax.numpy as jnp
from jax import lax
from jax.experimental import pallas as pl
from jax.experimental.pallas import tpu as pltpu
from jax.experimental.pallas import tpu_sc as plsc

_NS = 16
_NC = 2
_CH = 128
_EPC = _CH
_L = 16


def _seg_sum_call(table, srcp, dstp, NP):
  W = table.shape[1]
  NCH = srcp.shape[2]
  rpt = NP // _NS
  mesh = plsc.VectorSubcoreMesh(core_axis_name="c", subcore_axis_name="s")

  def body(table_h, src_h, dst_h, acc_out, src_v, dst_v, rows_v, zrow_v,
           acc_s, sem):
    cid = lax.axis_index("c")
    sid = lax.axis_index("s")
    base = sid * rpt

    for i in range(16):
      for j in range(W // _L):
        zrow_v[i, pl.ds(j * _L, _L)] = jnp.zeros((_L,), jnp.float32)

    def zloop(r, _):
      pltpu.sync_copy(zrow_v, acc_s.at[pl.ds(base + r * 16, 16)])
      return 0
    lax.fori_loop(0, rpt // 16, zloop, 0)
    plsc.subcore_barrier()

    def chunk(c, _):
      pltpu.sync_copy(src_h.at[cid, sid, c], src_v)
      pltpu.sync_copy(dst_h.at[cid, sid, c], dst_v)
      pltpu.async_copy(table_h.at[src_v], rows_v, sem).wait()
      pltpu.sync_copy(rows_v, acc_s.at[dst_v], add=True)
      return 0
    lax.fori_loop(0, NCH, chunk, 0)
    plsc.subcore_barrier()

    pltpu.sync_copy(acc_s.at[pl.ds(base, rpt)],
                    acc_out.at[cid, pl.ds(base, rpt)])

  fn = pl.kernel(
      body,
      out_type=jax.ShapeDtypeStruct((2, NP, W), jnp.float32),
      mesh=mesh,
      scratch_types=(
          pltpu.VMEM(srcp.shape[3:], jnp.int32),
          pltpu.VMEM(srcp.shape[3:], jnp.int32),
          pltpu.VMEM((_EPC, W), jnp.float32),

          pltpu.VMEM((16, W), jnp.float32),
          pltpu.VMEM_SHARED((NP, W), jnp.float32),
          pltpu.SemaphoreType.DMA,
      ),
  )
  return fn(table, srcp, dstp)


def _count_call(dstp, NP):
  NCH = dstp.shape[2]
  mesh = plsc.VectorSubcoreMesh(core_axis_name="c", subcore_axis_name="s")

  def body(dst_h, cnt_out, dst_v, cnt_v):
    cid = lax.axis_index("c")
    sid = lax.axis_index("s")

    def z1(r, _):
      cnt_v[pl.ds(r * _L, _L)] = jnp.zeros((_L,), jnp.float32)
      return 0
    lax.fori_loop(0, NP // _L, z1, 0)

    ones16 = jnp.ones((_L,), jnp.float32)

    def chunk(c, _):
      pltpu.sync_copy(dst_h.at[cid, sid, c], dst_v)
      for j in range(_CH // _L):
        idxv = dst_v[pl.ds(j * _L, _L)]
        plsc.addupdate_scatter(cnt_v, [idxv], ones16)
      return 0
    lax.fori_loop(0, NCH, chunk, 0)

    pltpu.sync_copy(cnt_v, cnt_out.at[cid, sid])

  fn = pl.kernel(
      body,
      out_type=jax.ShapeDtypeStruct((2, _NS, NP), jnp.float32),
      mesh=mesh,
      compiler_params=pltpu.CompilerParams(needs_layout_passes=False),
      scratch_types=(
          pltpu.VMEM((_CH,), jnp.int32),
          pltpu.VMEM((NP,), jnp.float32),
      ),
  )
  return fn(dstp)


def _tc_layer1(acc, cnt, xin, wlT, bl, wrT, R=1024):
  G, NP, D = acc.shape

  def body(acc_r, cnt_r, x_r, wl_r, bl_r, wr_r, out_r):
    c = jnp.maximum(jnp.sum(cnt_r[0], axis=0), 1.0)[:, None]
    h = (jnp.dot(acc_r[0] / c, wl_r[...], preferred_element_type=jnp.float32)
         + jnp.dot(x_r[0], wr_r[...], preferred_element_type=jnp.float32)
         + bl_r[...])
    out_r[0] = jnp.maximum(h, 0.0)

  return pl.pallas_call(
      body,
      grid=(G, NP // R),
      in_specs=[
          pl.BlockSpec((1, R, D), lambda g, i: (g, i, 0)),
          pl.BlockSpec((1, _NS, R), lambda g, i: (g, 0, i)),
          pl.BlockSpec((1, R, D), lambda g, i: (g, i, 0)),
          pl.BlockSpec((D, D), lambda g, i: (0, 0)),
          pl.BlockSpec((1, D), lambda g, i: (0, 0)),
          pl.BlockSpec((D, D), lambda g, i: (0, 0)),
      ],
      out_specs=pl.BlockSpec((1, R, D), lambda g, i: (g, i, 0)),
      out_shape=jax.ShapeDtypeStruct((G, NP, D), jnp.float32),
  )(acc, cnt, xin, wlT, bl, wrT)


def _tc_layer2(acc, cnt, xz1, wlT, bl, wrT, w8, b8, R=1024):
  G, NP, D = acc.shape

  def body(acc_r, cnt_r, x_r, wl_r, bl_r, wr_r, w8_r, b8_r, out_r):
    c = jnp.maximum(jnp.sum(cnt_r[0], axis=0), 1.0)[:, None]
    xz2 = (jnp.dot(acc_r[0] / c, wl_r[...], preferred_element_type=jnp.float32)
           + jnp.dot(x_r[0], wr_r[...], preferred_element_type=jnp.float32)
           + bl_r[...])
    hd = lax.dot_general(w8_r[...], xz2, (((1,), (1,)), ((), ())),
                         preferred_element_type=jnp.float32) + b8_r[...]
    out_r[0] = jnp.where(hd >= 0.0, hd, 0.01 * hd)

  return pl.pallas_call(
      body,
      grid=(G, NP // R),
      in_specs=[
          pl.BlockSpec((1, R, D), lambda g, i: (g, i, 0)),
          pl.BlockSpec((1, _NS, R), lambda g, i: (g, 0, i)),
          pl.BlockSpec((1, R, D), lambda g, i: (g, i, 0)),
          pl.BlockSpec((D, D), lambda g, i: (0, 0)),
          pl.BlockSpec((1, D), lambda g, i: (0, 0)),
          pl.BlockSpec((D, D), lambda g, i: (0, 0)),
          pl.BlockSpec((8, D), lambda g, i: (0, 0)),
          pl.BlockSpec((8, 1), lambda g, i: (0, 0)),
      ],
      out_specs=pl.BlockSpec((1, 8, R), lambda g, i: (g, 0, i)),
      out_shape=jax.ShapeDtypeStruct((G, 8, NP), jnp.float32),
  )(acc, cnt, xz1, wlT, bl, wrT, w8, b8)


def _gather_heads(hv, maskp, tp, cp):
  H4, NPad = hv.shape
  MP = maskp.shape[0]
  TP = tp.shape[0]
  mesh = plsc.VectorSubcoreMesh(core_axis_name="c", subcore_axis_name="s")

  def body(hv_h, mask_h, t_h, c_h, out_h, hv_v, mask_v, t_v, c_v, out_v):
    cid = lax.axis_index("c")
    sid = lax.axis_index("s")

    pltpu.sync_copy(hv_h, hv_v)
    pltpu.sync_copy(mask_h, mask_v)
    pltpu.sync_copy(t_h, t_v)
    pltpu.sync_copy(c_h, c_v)
    for i in range(TP // _L):
      sl = pl.ds(i * _L, _L)
      rt = plsc.load_gather(mask_v, [t_v[sl]])
      rc = plsc.load_gather(mask_v, [c_v[sl]])
      for k, r in ((0, rt), (1, rt), (2, rc), (3, rc)):
        row = jnp.full((_L,), k, jnp.int32)
        out_v[k, sl] = plsc.load_gather(hv_v, [row, r])

    @pl.when(jnp.logical_and(cid == 0, sid == 0))
    def _():
      pltpu.sync_copy(out_v, out_h)

  fn = pl.kernel(
      body,
      out_type=jax.ShapeDtypeStruct((4, TP), jnp.float32),
      mesh=mesh,
      compiler_params=pltpu.CompilerParams(needs_layout_passes=False),
      scratch_types=(
          pltpu.VMEM((H4, NPad), jnp.float32),
          pltpu.VMEM((MP,), jnp.int32),
          pltpu.VMEM((TP,), jnp.int32),
          pltpu.VMEM((TP,), jnp.int32),
          pltpu.VMEM((4, TP), jnp.float32),
      ),
  )
  return fn(hv, maskp, tp, cp)


def kernel(x, edge_index, fake_x, fake_edge_index, mask, treat_idx, control_idx,
           Wl1, bl1, Wr1, Wl2, bl2, Wr2, Wy1, by1, Wy0, by0, Wbal, bbal,
           Wprop, bprop):
  N, D = x.shape
  E = edge_index.shape[1]
  f32 = jnp.float32
  i32 = jnp.int32

  NP = -(-(N + 1) // 256) * 256
  EPT = -(-E // _NS // _CH) * _CH
  EP = EPT * _NS
  NCH = EPT // _CH

  xpad = jnp.zeros((2 * NP, D), f32).at[:N].set(x).at[NP:NP + N].set(fake_x)

  pad_src = jnp.zeros((EP - E,), i32)
  pad_dst = N + (jnp.arange(EP - E, dtype=i32) % (NP - N))
  src2 = jnp.stack([jnp.concatenate([edge_index[0], pad_src]),
                    jnp.concatenate([fake_edge_index[0] + NP, pad_src])])
  dst2 = jnp.stack([jnp.concatenate([edge_index[1], pad_dst]),
                    jnp.concatenate([fake_edge_index[1], pad_dst])])

  srcp = src2.reshape(2, _NS, NCH, _CH)
  dstp = dst2.reshape(2, _NS, NCH, _CH)

  acc1 = _seg_sum_call(xpad, srcp, dstp, NP)
  cnt = _count_call(dstp, NP)
  xz1 = _tc_layer1(acc1, cnt, xpad.reshape(2, NP, D),
                   Wl1.T, bl1.reshape(1, D), Wr1.T)
  acc2 = _seg_sum_call(xz1.reshape(2 * NP, D), srcp, dstp, NP)

  w8 = jnp.concatenate([Wy1, Wy0, Wprop, Wbal, jnp.zeros((4, D), f32)], 0)
  b8 = jnp.concatenate([by1, by0, bprop, bbal,
                        jnp.zeros((4,), f32)]).reshape(8, 1)
  heads = _tc_layer2(acc2, cnt, xz1, Wl2.T, bl2.reshape(1, D), Wr2.T, w8, b8)

  hv = jnp.stack([heads[0, 0], heads[1, 0], heads[0, 1], heads[1, 1]])

  T = treat_idx.shape[0]
  M = mask.shape[0]
  TP = -(-T // _CH) * _CH
  MP = -(-M // _CH) * _CH
  maskp = jnp.concatenate([mask, jnp.zeros((MP - M,), i32)])
  tpad = jnp.concatenate([treat_idx, jnp.zeros((TP - T,), i32)])
  cpad = jnp.concatenate([control_idx, jnp.zeros((TP - T,), i32)])
  g4 = _gather_heads(hv, maskp, tpad, cpad)

  return (g4[0, :T], g4[1, :T], g4[2, :T], g4[3, :T],
          heads[0, 2, :N], heads[1, 2, :N], heads[0, 3, :N])
